# baseline (speedup 1.0000x reference)
import functools

import jax
import jax.numpy as jnp
from jax import lax
from jax.experimental import pallas as pl
from jax.experimental.pallas import tpu as pltpu
from jax.experimental.pallas import tpu_sc as plsc

_BINS = 256
_LANES = 16


def _sc_partial_hists(x4, s_off, nsamp):
    info = plsc.get_sparse_core_info()
    nc, ns = info.num_cores, info.num_subcores
    nw = nc * ns
    _, ch, h, w = x4.shape
    wps = nw // nsamp
    ch_w = ch // wps
    assert wps * nsamp == nw and ch_w * wps == ch and ch_w % 2 == 0
    npairs = ch_w // 2

    mesh = plsc.VectorSubcoreMesh(core_axis_name="c", subcore_axis_name="s")

    @functools.partial(
        pl.kernel,
        mesh=mesh,
        out_type=jax.ShapeDtypeStruct((nw, _BINS), jnp.float32),
        compiler_params=pltpu.CompilerParams(needs_layout_passes=False),
        scratch_types=[
            pltpu.VMEM((h, w), jnp.float32),
            pltpu.VMEM((h, w), jnp.float32),
            pltpu.VMEM((_LANES * _BINS,), jnp.float32),
            pltpu.VMEM((_BINS,), jnp.float32),
            pltpu.SemaphoreType.DMA,
            pltpu.SemaphoreType.DMA,
        ],
    )
    def hist_kernel(x_hbm, out_hbm, buf0, buf1, hist, outv, sem0, sem1):
        c = lax.axis_index("c")
        s = lax.axis_index("s")
        wid = c * ns + s
        samp = s_off + wid // wps
        c_base = (wid % wps) * ch_w

        zero16 = jnp.zeros((_LANES,), jnp.float32)
        ones16 = jnp.ones((_LANES,), jnp.float32)
        lane_iota = lax.iota(jnp.int32, _LANES)

        def zbody(i, carry):
            hist[pl.ds(i * _LANES, _LANES)] = zero16
            return carry

        lax.fori_loop(0, _BINS, zbody, 0)

        def issue(g, buf, sem):
            return pltpu.async_copy(x_hbm.at[samp, c_base + g], buf, sem)

        def wait(buf, sem):
            pltpu.make_async_copy(x_hbm.at[samp, c_base], buf, sem).wait()

        def process(buf):
            @plsc.parallel_loop(0, h, 1, unroll=2)
            def vbody(r):
                for l in range(w // _LANES):
                    val = buf[r, pl.ds(l * _LANES, _LANES)]
                    idx = (val * 256.0).astype(jnp.int32)
                    addr = (idx << 4) | lane_iota
                    plsc.addupdate_scatter(hist, [addr], ones16)

        issue(0, buf0, sem0)

        def pair(t, carry):
            g = 2 * t
            issue(g + 1, buf1, sem1)
            wait(buf0, sem0)
            process(buf0)

            @pl.when(g + 2 < ch_w)
            def _():
                issue(g + 2, buf0, sem0)

            wait(buf1, sem1)
            process(buf1)
            return carry

        lax.fori_loop(0, npairs, pair, 0)

        for j in range(_BINS // _LANES):
            acc = zero16
            for k in range(_LANES):
                sv = jnp.sum(hist[pl.ds((j * _LANES + k) * _LANES, _LANES)])
                acc = jnp.where(lane_iota == k, sv, acc)
            outv[pl.ds(j * _LANES, _LANES)] = acc
        pltpu.sync_copy(outv, out_hbm.at[wid])

    return hist_kernel(x4)


def _tc_mlp_scale(x, partials, W1, b1, W2, b2, s_off, nsamp, wps, prev=None):
    b, ch, h, w = x.shape
    cblk = 8
    jblk = ch // cblk
    assert jblk * cblk == ch

    parts3 = partials.reshape(nsamp, 1, wps * _BINS)

    def body(part_ref, w1_ref, b1_ref, w2_ref, b2_ref, x_ref, *rest):
        o_ref, wscr = rest[-2], rest[-1]

        @pl.when(pl.program_id(1) == 0)
        def _():
            hp = part_ref[0]
            hs = hp[:, :_BINS]
            for k in range(1, wps):
                hs = hs + hp[:, k * _BINS:(k + 1) * _BINS]
            y = jnp.dot(hs, w1_ref[...], preferred_element_type=jnp.float32)
            y = jnp.maximum(y + b1_ref[...], 0.0)
            wv = jnp.dot(y, w2_ref[...], preferred_element_type=jnp.float32)
            wscr[0, 0] = wv[0, 0] + b2_ref[0, 0]

        o_ref[...] = x_ref[...] * wscr[0, 0]

    in_specs = [
        pl.BlockSpec((1, 1, wps * _BINS), lambda i, j: (i, 0, 0)),
        pl.BlockSpec((_BINS, 16), lambda i, j: (0, 0)),
        pl.BlockSpec((1, 16), lambda i, j: (0, 0)),
        pl.BlockSpec((16, 1), lambda i, j: (0, 0)),
        pl.BlockSpec((1, 1), lambda i, j: (0, 0)),
        pl.BlockSpec((1, cblk, h, w), lambda i, j, s=s_off: (i + s, j, 0, 0)),
    ]
    ins = [parts3, W1, b1.reshape(1, 16), W2, b2.reshape(1, 1), x]
    aliases = {}
    if prev is not None:
        in_specs.append(pl.BlockSpec(memory_space=pl.ANY))
        ins.append(prev)
        aliases = {6: 0}

    return pl.pallas_call(
        body,
        grid=(nsamp, jblk),
        in_specs=in_specs,
        out_specs=pl.BlockSpec(
            (1, cblk, h, w), lambda i, j, s=s_off: (i + s, j, 0, 0)
        ),
        out_shape=jax.ShapeDtypeStruct((b, ch, h, w), jnp.float32),
        scratch_shapes=[pltpu.SMEM((1, 1), jnp.float32)],
        input_output_aliases=aliases,
    )(*ins)


def kernel(x, W1, b1, W2, b2):
    sizes = (2, 4, 4, 4, 2)
    offs, acc = [], 0
    for n in sizes:
        offs.append(acc)
        acc += n
    assert acc == x.shape[0]
    parts = [
        _sc_partial_hists(x, o, n) for o, n in zip(offs, sizes)
    ]
    out = None
    for p, o, n in zip(parts, offs, sizes):
        out = _tc_mlp_scale(x, p, W1, b1, W2, b2, o, n, 32 // n, prev=out)
    return out


# speedup vs baseline: 1.0249x; 1.0249x over previous
import functools

import jax
import jax.numpy as jnp
from jax import lax
from jax.experimental import pallas as pl
from jax.experimental.pallas import tpu as pltpu
from jax.experimental.pallas import tpu_sc as plsc

_BINS = 256
_LANES = 16


def _sc_partial_hists(x4, s_off, nsamp):
    info = plsc.get_sparse_core_info()
    nc, ns = info.num_cores, info.num_subcores
    nw = nc * ns
    _, ch, h, w = x4.shape
    wps = nw // nsamp
    ch_w = ch // wps
    assert wps * nsamp == nw and ch_w * wps == ch and ch_w % 2 == 0
    npairs = ch_w // 2

    mesh = plsc.VectorSubcoreMesh(core_axis_name="c", subcore_axis_name="s")

    @functools.partial(
        pl.kernel,
        mesh=mesh,
        out_type=jax.ShapeDtypeStruct((nw, _BINS), jnp.float32),
        compiler_params=pltpu.CompilerParams(needs_layout_passes=False),
        scratch_types=[
            pltpu.VMEM((h, w), jnp.float32),
            pltpu.VMEM((h, w), jnp.float32),
            pltpu.VMEM((_LANES * _BINS,), jnp.float32),
            pltpu.VMEM((_BINS,), jnp.float32),
            pltpu.SemaphoreType.DMA,
            pltpu.SemaphoreType.DMA,
        ],
    )
    def hist_kernel(x_hbm, out_hbm, buf0, buf1, hist, outv, sem0, sem1):
        c = lax.axis_index("c")
        s = lax.axis_index("s")
        wid = c * ns + s
        samp = s_off + wid // wps
        c_base = (wid % wps) * ch_w

        zero16 = jnp.zeros((_LANES,), jnp.float32)
        ones16 = jnp.ones((_LANES,), jnp.float32)
        lane_iota = lax.iota(jnp.int32, _LANES)

        def zbody(i, carry):
            hist[pl.ds(i * _LANES, _LANES)] = zero16
            return carry

        lax.fori_loop(0, _BINS, zbody, 0)

        def issue(g, buf, sem):
            return pltpu.async_copy(x_hbm.at[samp, c_base + g], buf, sem)

        def wait(buf, sem):
            pltpu.make_async_copy(x_hbm.at[samp, c_base], buf, sem).wait()

        def process(buf):
            @plsc.parallel_loop(0, h, 1, unroll=4)
            def vbody(r):
                for l in range(w // _LANES):
                    val = buf[r, pl.ds(l * _LANES, _LANES)]
                    idx = (val * 256.0).astype(jnp.int32)
                    addr = (idx << 4) | lane_iota
                    plsc.addupdate_scatter(hist, [addr], ones16)

        issue(0, buf0, sem0)

        def pair(t, carry):
            g = 2 * t
            issue(g + 1, buf1, sem1)
            wait(buf0, sem0)
            process(buf0)

            @pl.when(g + 2 < ch_w)
            def _():
                issue(g + 2, buf0, sem0)

            wait(buf1, sem1)
            process(buf1)
            return carry

        lax.fori_loop(0, npairs, pair, 0)

        for j in range(_BINS // _LANES):
            acc = zero16
            for k in range(_LANES):
                sv = jnp.sum(hist[pl.ds((j * _LANES + k) * _LANES, _LANES)])
                acc = jnp.where(lane_iota == k, sv, acc)
            outv[pl.ds(j * _LANES, _LANES)] = acc
        pltpu.sync_copy(outv, out_hbm.at[wid])

    return hist_kernel(x4)


def _tc_mlp_scale(x, partials, W1, b1, W2, b2, s_off, nsamp, wps, prev=None):
    b, ch, h, w = x.shape
    cblk = 8
    jblk = ch // cblk
    assert jblk * cblk == ch

    parts3 = partials.reshape(nsamp, 1, wps * _BINS)

    def body(part_ref, w1_ref, b1_ref, w2_ref, b2_ref, x_ref, *rest):
        o_ref, wscr = rest[-2], rest[-1]

        @pl.when(pl.program_id(1) == 0)
        def _():
            hp = part_ref[0]
            hs = hp[:, :_BINS]
            for k in range(1, wps):
                hs = hs + hp[:, k * _BINS:(k + 1) * _BINS]
            y = jnp.dot(hs, w1_ref[...], preferred_element_type=jnp.float32)
            y = jnp.maximum(y + b1_ref[...], 0.0)
            wv = jnp.dot(y, w2_ref[...], preferred_element_type=jnp.float32)
            wscr[0, 0] = wv[0, 0] + b2_ref[0, 0]

        o_ref[...] = x_ref[...] * wscr[0, 0]

    in_specs = [
        pl.BlockSpec((1, 1, wps * _BINS), lambda i, j: (i, 0, 0)),
        pl.BlockSpec((_BINS, 16), lambda i, j: (0, 0)),
        pl.BlockSpec((1, 16), lambda i, j: (0, 0)),
        pl.BlockSpec((16, 1), lambda i, j: (0, 0)),
        pl.BlockSpec((1, 1), lambda i, j: (0, 0)),
        pl.BlockSpec((1, cblk, h, w), lambda i, j, s=s_off: (i + s, j, 0, 0)),
    ]
    ins = [parts3, W1, b1.reshape(1, 16), W2, b2.reshape(1, 1), x]
    aliases = {}
    if prev is not None:
        in_specs.append(pl.BlockSpec(memory_space=pl.ANY))
        ins.append(prev)
        aliases = {6: 0}

    return pl.pallas_call(
        body,
        grid=(nsamp, jblk),
        in_specs=in_specs,
        out_specs=pl.BlockSpec(
            (1, cblk, h, w), lambda i, j, s=s_off: (i + s, j, 0, 0)
        ),
        out_shape=jax.ShapeDtypeStruct((b, ch, h, w), jnp.float32),
        scratch_shapes=[pltpu.SMEM((1, 1), jnp.float32)],
        input_output_aliases=aliases,
    )(*ins)


def kernel(x, W1, b1, W2, b2):
    b = x.shape[0]
    nsplit = 4
    nsamp = b // nsplit
    wps = 32 // nsamp
    parts = [
        _sc_partial_hists(x, q * nsamp, nsamp) for q in range(nsplit)
    ]
    out = None
    for q in range(nsplit):
        out = _tc_mlp_scale(
            x, parts[q], W1, b1, W2, b2, q * nsamp, nsamp, wps, prev=out
        )
    return out
